# Initial kernel scaffold; baseline (speedup 1.0000x reference)
import jax
import jax.numpy as jnp
from jax.experimental import pallas as pl


def kernel(input, params):
    raise NotImplementedError("write your pallas kernel here")


# speedup vs baseline: 1.0001x; 1.0001x over previous
import jax, jax.numpy as jnp
import numpy as np
from jax.experimental import pallas as pl

_K = 16
_D = 4

def _knn(ref, query, k, chunk=2048):
    r2 = jnp.sum(ref ** 2, axis=-1)
    Nq = query.shape[1]
    idxs, dists = [], []
    for s in range(0, Nq, chunk):
        q = query[:, s:s + chunk]
        d2 = jnp.sum(q ** 2, axis=-1)[:, :, None] + r2[:, None, :] - 2.0 * jnp.einsum('bqd,brd->bqr', q, ref)
        neg, idx = jax.lax.top_k(-d2, k)
        idxs.append(idx)
        dists.append(jnp.sqrt(jnp.maximum(-neg, 1e-12)))
    return jnp.concatenate(idxs, axis=1), jnp.concatenate(dists, axis=1)

def _bn(x, p, eps=1e-6):
    mean = jnp.mean(x, axis=(0, 2, 3), keepdims=True)
    var = jnp.var(x, axis=(0, 2, 3), keepdims=True)
    xh = (x - mean) / jnp.sqrt(var + eps)
    return p['g'][None, :, None, None] * xh + p['be'][None, :, None, None]

def _smlp(x, p, bn=False, act=None):
    y = jnp.einsum('bcnk,oc->bonk', x, p['W']) + p['b'][None, :, None, None]
    if bn:
        y = _bn(y, p)
    if act is not None:
        y = act(y)
    return y

def _att_pool(x, p):
    xp = jnp.transpose(x, (0, 2, 3, 1))
    t = jnp.einsum('bnkc,oc->bnko', xp, p['score_W'])
    sc = jax.nn.softmax(t, axis=-2)
    sc = jnp.transpose(sc, (0, 3, 1, 2))
    feat = jnp.sum(sc * x, axis=-1, keepdims=True)
    return _smlp(feat, p['mlp'], bn=True, act=jax.nn.relu)

def _lse(coords, features, idx, dist, p):
    B, N, K = idx.shape
    bidx = jnp.arange(B)[:, None, None]
    nb = coords[bidx, idx]
    neighbors = jnp.transpose(nb, (0, 3, 1, 2))
    ext = jnp.transpose(coords, (0, 2, 1))[:, :, :, None]
    ext = jnp.broadcast_to(ext, (B, 3, N, K))
    concat = jnp.concatenate([ext, neighbors, ext - neighbors, dist[:, None, :, :]], axis=1)
    enc = _smlp(concat, p, bn=True, act=jax.nn.relu)
    featsb = jnp.broadcast_to(features, (B, features.shape[1], N, K))
    return jnp.concatenate([enc, featsb], axis=1)

def _lfa(coords, features, p, k):
    idx, dist = _knn(coords, coords, k)
    x = _smlp(features, p['mlp1'], act=lambda v: jax.nn.leaky_relu(v, 0.2))
    x = _lse(coords, x, idx, dist, p['lse1'])
    x = _att_pool(x, p['pool1'])
    x = _lse(coords, x, idx, dist, p['lse2'])
    x = _att_pool(x, p['pool2'])
    y = _smlp(x, p['mlp2']) + _smlp(features, p['shortcut'], bn=True)
    return jax.nn.leaky_relu(y, 0.01)

def kernel(input, params):
    inp = input
    B, N, _ = inp.shape
    d = _D
    coords = inp[..., :3]
    x = inp @ params['fc_start']['W'].T + params['fc_start']['b']
    x = jnp.transpose(x, (0, 2, 1))[:, :, :, None]
    x = _bn(x, params['bn_start'])
    x = jax.nn.leaky_relu(x, 0.2)
    perm = jax.random.permutation(jax.random.key(1), N)
    coords = coords[:, perm]
    x = x[:, :, perm]
    dr = 1
    stack = []
    for p in params['enc']:
        x = _lfa(coords[:, :N // dr], x, p, _K)
        stack.append(x)
        dr = dr * d
        x = x[:, :, :N // dr]
    x = _smlp(x, params['mid'], act=jax.nn.relu)
    for p in params['dec']:
        idx, _ = _knn(coords[:, :N // dr], coords[:, :d * N // dr], 1)
        xg = x[:, :, :, 0]
        xn = jnp.take_along_axis(xg, idx[:, :, 0][:, None, :], axis=2)[:, :, :, None]
        x = jnp.concatenate([xn, stack.pop()], axis=1)
        x = _smlp(x, p, bn=True, act=jax.nn.relu)
        dr = dr // d
    x = x[:, :, jnp.argsort(perm)]
    x = _smlp(x, params['fc_end'][0], bn=True, act=jax.nn.relu)
    x = _smlp(x, params['fc_end'][1], bn=True, act=jax.nn.relu)
    x = _smlp(x, params['fc_end'][2])
    return x[:, :, :, 0]


# speedup vs baseline: 7.6798x; 7.6792x over previous
import functools

import jax
import jax.numpy as jnp
from jax import lax
from jax.experimental import pallas as pl
from jax.experimental.pallas import tpu as pltpu
from jax.experimental.pallas import tpu_sc as plsc

NK = 16
DEC = 4
EPS = 1e-6
HI = lax.Precision.HIGHEST

_pcall = pl.pallas_call


def _dot(a, b):
    return lax.dot_general(a.astype(jnp.bfloat16), b.astype(jnp.bfloat16),
                           (((1,), (0,)), ((), ())),
                           preferred_element_type=jnp.float32)


def _leaky(x, s):
    return jnp.where(x >= 0, x, s * x)


def _stats_of(y, axes):
    return jnp.sum(y, axis=axes), jnp.sum(y * y, axis=axes)


def _affine_from_stats(stats, g, be, cnt):
    tot = jnp.sum(stats, axis=0)
    mean = tot[0] / cnt
    var = tot[1] / cnt - mean * mean
    scale = g / jnp.sqrt(var + EPS)
    shift = be - mean * scale
    return scale[:, None], shift[:, None]


def _d2_refstyle(cT, qT):
    r2 = jnp.sum(cT * cT, axis=0, keepdims=True)
    q2 = jnp.sum(qT * qT, axis=0, keepdims=True)
    prod = lax.dot_general(qT.astype(jnp.bfloat16), cT.astype(jnp.bfloat16),
                           (((0,), (0,)), ((), ())),
                           preferred_element_type=jnp.float32)
    return jnp.transpose(q2) + r2 - 2.0 * prod


def _knn_body(k, n, cref, qref, idx_ref, dist_ref):
    d2 = _d2_refstyle(cref[0], qref[0])
    bq = d2.shape[0]
    iota = lax.broadcasted_iota(jnp.int32, (bq, n), 1)
    for j in range(k):
        m = jnp.min(d2, axis=1, keepdims=True)
        sel = jnp.where(d2 == m, iota, n)
        ij = jnp.min(sel, axis=1, keepdims=True)
        dist_ref[0, j, :] = jnp.sqrt(jnp.maximum(m, 1e-12))[:, 0]
        idx_ref[0, j, :] = ij[:, 0]
        d2 = jnp.where(iota == ij, jnp.float32(jnp.inf), d2)


def _knn_topk(coordsT, k):
    B, _, n = coordsT.shape
    bq = min(n, 512)
    grid = (B, n // bq)
    return _pcall(
        functools.partial(_knn_body, k, n),
        grid=grid,
        in_specs=[
            pl.BlockSpec((1, 3, n), lambda b, j: (b, 0, 0)),
            pl.BlockSpec((1, 3, bq), lambda b, j: (b, 0, j)),
        ],
        out_specs=[
            pl.BlockSpec((1, k, bq), lambda b, j: (b, 0, j)),
            pl.BlockSpec((1, k, bq), lambda b, j: (b, 0, j)),
        ],
        out_shape=[
            jax.ShapeDtypeStruct((B, k, n), jnp.int32),
            jax.ShapeDtypeStruct((B, k, n), jnp.float32),
        ],
    )(coordsT, coordsT)


def _argmin_body(nr, cref, qref, idx_ref):
    d2 = _d2_refstyle(cref[0], qref[0])
    bq = d2.shape[0]
    iota = lax.broadcasted_iota(jnp.int32, (bq, nr), 1)
    m = jnp.min(d2, axis=1, keepdims=True)
    ij = jnp.min(jnp.where(d2 == m, iota, nr), axis=1, keepdims=True)
    idx_ref[0, 0, :] = ij[:, 0]


def _argmin_nn(refT, queryT):
    B, _, nr = refT.shape
    nq = queryT.shape[2]
    bq = min(nq, 1024)
    out = _pcall(
        functools.partial(_argmin_body, nr),
        grid=(B, nq // bq),
        in_specs=[
            pl.BlockSpec((1, 3, nr), lambda b, j: (b, 0, 0)),
            pl.BlockSpec((1, 3, bq), lambda b, j: (b, 0, j)),
        ],
        out_specs=pl.BlockSpec((1, 1, bq), lambda b, j: (b, 0, j)),
        out_shape=jax.ShapeDtypeStruct((B, 1, nq), jnp.int32),
    )(refT, queryT)
    return out[:, 0, :]


def _gather_rows(table, flat_idx):
    M = flat_idx.shape[0]
    D = table.shape[1]
    Dp = ((D + 127) // 128) * 128
    if Dp != D:
        table = jnp.pad(table, ((0, 0), (0, Dp - D)))
    mesh = plsc.VectorSubcoreMesh(core_axis_name="c", subcore_axis_name="s")
    NW = 32
    Mp = M if M % (8 * NW) == 0 else ((M // (8 * NW)) + 1) * (8 * NW)
    if Mp != M:
        flat_idx = jnp.concatenate(
            [flat_idx, jnp.zeros((Mp - M,), jnp.int32)])
    b_per_w = Mp // NW
    chunk = min(128, b_per_w)
    nchunks = b_per_w // chunk
    idx2d = flat_idx.reshape(Mp // chunk, chunk)

    @functools.partial(
        pl.kernel, mesh=mesh,
        out_type=jax.ShapeDtypeStruct((Mp, Dp), jnp.float32),
        scratch_types=[
            pltpu.VMEM((nchunks, chunk), jnp.int32),
            pltpu.VMEM((chunk, Dp), jnp.float32),
            pltpu.VMEM((chunk, Dp), jnp.float32),
            pltpu.SemaphoreType.DMA,
            pltpu.SemaphoreType.DMA,
        ])
    def _k(tab_hbm, i_hbm, o_hbm, idx_v, r0, r1, s0, s1):
        wid = lax.axis_index("s") * 2 + lax.axis_index("c")
        base = wid * b_per_w
        pltpu.sync_copy(i_hbm.at[pl.ds(wid * nchunks, nchunks)], idx_v)
        if nchunks == 1:
            pltpu.async_copy(tab_hbm.at[idx_v.at[0]], r0, s0).wait()
            pltpu.sync_copy(r0, o_hbm.at[pl.ds(base, chunk)])
        else:
            pltpu.async_copy(tab_hbm.at[idx_v.at[0]], r0, s0)
            pltpu.async_copy(tab_hbm.at[idx_v.at[1]], r1, s1)

            @pl.loop(0, nchunks, step=2)
            def _(i):
                pltpu.make_async_copy(
                    o_hbm.at[pl.ds(base, chunk)], r0, s0).wait()
                pltpu.sync_copy(r0, o_hbm.at[pl.ds(base + i * chunk, chunk)])

                @pl.when(i + 2 < nchunks)
                def _():
                    pltpu.async_copy(tab_hbm.at[idx_v.at[i + 2]], r0, s0)

                pltpu.make_async_copy(
                    o_hbm.at[pl.ds(base, chunk)], r1, s1).wait()
                pltpu.sync_copy(
                    r1, o_hbm.at[pl.ds(base + (i + 1) * chunk, chunk)])

                @pl.when(i + 3 < nchunks)
                def _():
                    pltpu.async_copy(tab_hbm.at[idx_v.at[i + 3]], r1, s1)

    out = _k(table, idx2d)
    if Mp != M or Dp != D:
        out = out[:M, :D]
    return out


def _gather_batched(tblT, idx):
    B, c, R = tblT.shape
    table = tblT.transpose(0, 2, 1).reshape(B * R, c)
    off = (jnp.arange(B, dtype=jnp.int32) * R).reshape((B,) + (1,) * (idx.ndim - 1))
    flat = (idx + off).reshape(-1)
    rows = _gather_rows(table, flat)
    out = rows.reshape(idx.shape + (c,))
    perm = (0, idx.ndim) + tuple(range(1, idx.ndim))
    return out.transpose(perm)


def _enc_a_body(finalize, slope, fref, W1, b1, Wsc, bsc, fscale, fshift,
                x1_o, scp_o, st_o):
    feats = fref[0]
    if finalize:
        feats = _leaky(fscale[...] * feats + fshift[...], slope)
    x1_o[0] = _leaky(_dot(W1[...], feats) + b1[...], 0.2)
    scp = _dot(Wsc[...], feats) + bsc[...]
    scp_o[0] = scp
    s, ss = _stats_of(scp, (1,))
    st_o[0, 0, :] = s
    st_o[0, 1, :] = ss


def _enc_a(featsP, W1, b1, Wsc, bsc, fscale=None, fshift=None, slope=0.2):
    B, cin, n = featsP.shape
    d2 = W1.shape[0]
    c2 = Wsc.shape[0]
    finalize = fscale is not None
    if not finalize:
        fscale = jnp.zeros((cin, 1), jnp.float32)
        fshift = jnp.zeros((cin, 1), jnp.float32)
    full = lambda *s: pl.BlockSpec((1,) + s, lambda b: (b,) + (0,) * len(s))
    wspec = lambda a: pl.BlockSpec(a.shape, lambda b: (0,) * a.ndim)
    return _pcall(
        functools.partial(_enc_a_body, finalize, slope),
        grid=(B,),
        in_specs=[full(cin, n)] + [
            wspec(W1), wspec(b1), wspec(Wsc), wspec(bsc),
            wspec(fscale), wspec(fshift)],
        out_specs=[full(d2, n), full(c2, n), full(2, c2)],
        out_shape=[
            jax.ShapeDtypeStruct((B, d2, n), jnp.float32),
            jax.ShapeDtypeStruct((B, c2, n), jnp.float32),
            jax.ShapeDtypeStruct((B, 2, c2), jnp.float32),
        ],
    )(featsP, W1, b1, Wsc, bsc, fscale, fshift)


def _lse_body(cref, nb_r, dist_r, W_r, b_r, enc_o, st_o):
    ext = cref[0]
    W = W_r[...]
    b = b_r[...]
    K = nb_r.shape[2]
    ssum = None
    for k in range(K):
        nb = nb_r[0, :, k, :]
        dk = dist_r[0, k:k + 1, :]
        cc = jnp.concatenate([ext, nb, ext - nb, dk], axis=0)
        enc = _dot(W, cc) + b
        enc_o[0, :, k, :] = enc
        s, ss = enc, enc * enc
        if ssum is None:
            ssum, sssum = s, ss
        else:
            ssum, sssum = ssum + s, sssum + ss
    st_o[0, 0, :] = jnp.sum(ssum, axis=1)
    st_o[0, 1, :] = jnp.sum(sssum, axis=1)


def _lse(coordsT, nbg, dist, W, b):
    B, _, n = coordsT.shape
    d2 = W.shape[0]
    full = lambda *s: pl.BlockSpec((1,) + s, lambda b: (b,) + (0,) * len(s))
    wspec = lambda a: pl.BlockSpec(a.shape, lambda b: (0,) * a.ndim)
    return _pcall(
        _lse_body,
        grid=(B,),
        in_specs=[full(3, n), full(3, NK, n), full(NK, n),
                  wspec(W), wspec(b)],
        out_specs=[full(d2, NK, n), full(2, d2)],
        out_shape=[
            jax.ShapeDtypeStruct((B, d2, NK, n), jnp.float32),
            jax.ShapeDtypeStruct((B, 2, d2), jnp.float32),
        ],
    )(coordsT, nbg, dist, W, b)


def _pool_body(fin, enc_r, esc_r, esh_r, fp_r, fsc_r, fsh_r, scW_r,
               Wpa_r, Wpb_r, bp_r, p_o, st_o):
    esc = esc_r[...]
    esh = esh_r[...]
    fprev = fp_r[0]
    if fin:
        fprev = jnp.maximum(fsc_r[...] * fprev + fsh_r[...], 0.0)
    scW = scW_r[...]
    K = enc_r.shape[2]
    ench = []
    tks = []
    for k in range(K):
        e = jnp.maximum(esc * enc_r[0, :, k, :] + esh, 0.0)
        ench.append(e)
        tks.append(_dot(scW, e))
    m = tks[0]
    for k in range(1, K):
        m = jnp.maximum(m, tks[k])
    es = [jnp.exp(tks[k] - m) for k in range(K)]
    ssum = es[0]
    for k in range(1, K):
        ssum = ssum + es[k]
    attn = None
    for k in range(K):
        t = (es[k] / ssum) * ench[k]
        attn = t if attn is None else attn + t
    p = _dot(Wpa_r[...], attn) + _dot(Wpb_r[...], fprev) + bp_r[...]
    p_o[0] = p
    s, ss = _stats_of(p, (1,))
    st_o[0, 0, :] = s
    st_o[0, 1, :] = ss


def _pool(enc_pre, escale, eshift, fprev, scW, Wpa, Wpb, bp,
          fscale=None, fshift=None):
    B, d2, K, n = enc_pre.shape
    op = Wpa.shape[0]
    fin = fscale is not None
    if not fin:
        fscale = jnp.zeros((d2, 1), jnp.float32)
        fshift = jnp.zeros((d2, 1), jnp.float32)
    full = lambda *s: pl.BlockSpec((1,) + s, lambda b: (b,) + (0,) * len(s))
    wspec = lambda a: pl.BlockSpec(a.shape, lambda b: (0,) * a.ndim)
    return _pcall(
        functools.partial(_pool_body, fin),
        grid=(B,),
        in_specs=[full(d2, K, n), wspec(escale), wspec(eshift),
                  full(d2, n), wspec(fscale), wspec(fshift),
                  wspec(scW), wspec(Wpa), wspec(Wpb), wspec(bp)],
        out_specs=[full(op, n), full(2, op)],
        out_shape=[
            jax.ShapeDtypeStruct((B, op, n), jnp.float32),
            jax.ShapeDtypeStruct((B, 2, op), jnp.float32),
        ],
    )(enc_pre, escale, eshift, fprev, fscale, fshift, scW, Wpa, Wpb, bp)


def _enc_f_body(p2_r, psc_r, psh_r, Wm_r, bm_r, scp_r, ssc_r, ssh_r, out_o):
    p2 = jnp.maximum(psc_r[...] * p2_r[0] + psh_r[...], 0.0)
    y = _dot(Wm_r[...], p2) + bm_r[...] + ssc_r[...] * scp_r[0] + ssh_r[...]
    out_o[0] = _leaky(y, 0.01)


def _enc_f(p2_pre, pscale, pshift, Wm, bm, scp, sscale, sshift):
    B, d, n = p2_pre.shape
    c2 = Wm.shape[0]
    full = lambda *s: pl.BlockSpec((1,) + s, lambda b: (b,) + (0,) * len(s))
    wspec = lambda a: pl.BlockSpec(a.shape, lambda b: (0,) * a.ndim)
    return _pcall(
        _enc_f_body,
        grid=(B,),
        in_specs=[full(d, n), wspec(pscale), wspec(pshift),
                  wspec(Wm), wspec(bm), full(c2, n),
                  wspec(sscale), wspec(sshift)],
        out_specs=full(c2, n),
        out_shape=jax.ShapeDtypeStruct((B, c2, n), jnp.float32),
    )(p2_pre, pscale, pshift, Wm, bm, scp, sscale, sshift)


def _mlp_body(fin, act, stats, x_r, sc_r, sh_r, W1_r, b_r, skip_r, W2_r,
              y_o, st_o):
    x = x_r[0]
    if fin:
        x = jnp.maximum(sc_r[...] * x + sh_r[...], 0.0)
    y = _dot(W1_r[...], x) + b_r[...]
    if skip_r is not None:
        y = y + _dot(W2_r[...], skip_r[0])
    if act == "relu":
        y = jnp.maximum(y, 0.0)
    y_o[0] = y
    if stats:
        s, ss = _stats_of(y, (1,))
        st_o[0, 0, :] = s
        st_o[0, 1, :] = ss


def _mlp(x, W1, b, skip=None, W2=None, scale=None, shift=None, act=None,
         stats=False):
    B, cin, n = x.shape
    o = W1.shape[0]
    fin = scale is not None
    if not fin:
        scale = jnp.zeros((cin, 1), jnp.float32)
        shift = jnp.zeros((cin, 1), jnp.float32)
    full = lambda *s: pl.BlockSpec((1,) + s, lambda b: (b,) + (0,) * len(s))
    wspec = lambda a: pl.BlockSpec(a.shape, lambda b: (0,) * a.ndim)
    has_skip = skip is not None
    if not has_skip:
        skip = jnp.zeros((B, 8, n), jnp.float32)
        W2 = jnp.zeros((o, 8), jnp.float32)

    def body(x_r, sc_r, sh_r, W1_r, b_r, skip_r, W2_r, y_o, st_o):
        _mlp_body(fin, act, stats, x_r, sc_r, sh_r, W1_r, b_r,
                  skip_r if has_skip else None, W2_r, y_o, st_o)

    outs = _pcall(
        body,
        grid=(B,),
        in_specs=[full(cin, n), wspec(scale), wspec(shift),
                  wspec(W1), wspec(b), full(skip.shape[1], n), wspec(W2)],
        out_specs=[full(o, n), full(2, o)],
        out_shape=[
            jax.ShapeDtypeStruct((B, o, n), jnp.float32),
            jax.ShapeDtypeStruct((B, 2, o), jnp.float32),
        ],
    )(x, scale, shift, W1, b, skip, W2)
    return outs if stats else outs[0]


def _level(coordsT, featsP, p, fscale, fshift, slope):
    B, cin, n = featsP.shape
    d2 = p['mlp1']['W'].shape[0]
    d = 2 * d2

    idx, dist = _knn_topk(coordsT, NK)

    x1, scp, sc_st = _enc_a(
        featsP, p['mlp1']['W'], p['mlp1']['b'][:, None],
        p['shortcut']['W'], p['shortcut']['b'][:, None],
        fscale, fshift, slope)

    nbg = _gather_batched(coordsT, idx)

    enc1, st1 = _lse(coordsT, nbg, dist, p['lse1']['W'],
                     p['lse1']['b'][:, None])
    e1s, e1h = _affine_from_stats(st1, p['lse1']['g'], p['lse1']['be'],
                                  B * n * NK)
    scW1 = p['pool1']['score_W'][:d2, :d2]
    Wp1 = p['pool1']['mlp']['W']
    p1_pre, pst1 = _pool(enc1, e1s, e1h, x1, scW1,
                         Wp1[:, :d2], Wp1[:, d2:],
                         p['pool1']['mlp']['b'][:, None])
    p1s, p1h = _affine_from_stats(pst1, p['pool1']['mlp']['g'],
                                  p['pool1']['mlp']['be'], B * n)

    enc2, st2 = _lse(coordsT, nbg, dist, p['lse2']['W'],
                     p['lse2']['b'][:, None])
    e2s, e2h = _affine_from_stats(st2, p['lse2']['g'], p['lse2']['be'],
                                  B * n * NK)
    scW2 = p['pool2']['score_W'][:d2, :d2]
    Wp2 = p['pool2']['mlp']['W']
    p2_pre, pst2 = _pool(enc2, e2s, e2h, p1_pre, scW2,
                         Wp2[:, :d2], Wp2[:, d2:],
                         p['pool2']['mlp']['b'][:, None],
                         fscale=p1s, fshift=p1h)
    p2s, p2h = _affine_from_stats(pst2, p['pool2']['mlp']['g'],
                                  p['pool2']['mlp']['be'], B * n)

    scs, sch = _affine_from_stats(sc_st, p['shortcut']['g'],
                                  p['shortcut']['be'], B * n)
    out = _enc_f(p2_pre, p2s, p2h, p['mlp2']['W'],
                 p['mlp2']['b'][:, None], scp, scs, sch)
    return out


def kernel(input, params):
    inp = input
    B, N, _ = inp.shape
    nc = params['fc_end'][2]['W'].shape[0]

    inpT = inp.transpose(0, 2, 1)
    x0_pre, st0 = _mlp(inpT, params['fc_start']['W'],
                       params['fc_start']['b'][:, None], stats=True)
    f0s, f0h = _affine_from_stats(st0, params['bn_start']['g'],
                                  params['bn_start']['be'], B * N)

    perm = jax.random.permutation(jax.random.key(1), N)
    coordsT = inpT[:, :3, :][:, :, perm]
    x0_pre = x0_pre[:, :, perm]

    dr = 1
    x = x0_pre
    fscale, fshift, slope = f0s, f0h, 0.2
    stack = []
    for li, p in enumerate(params['enc']):
        n = N // dr
        out = _level(coordsT[:, :, :n], x[:, :, :n] if li == 0 else x,
                     p, fscale, fshift, slope)
        stack.append(out)
        dr *= DEC
        x = out[:, :, :N // dr]
        fscale = fshift = None
        slope = None

    x = _mlp(x, params['mid']['W'], params['mid']['b'][:, None], act="relu")

    dscale = dshift = None
    for p in params['dec']:
        n_small = N // dr
        n_big = DEC * N // dr
        idx = _argmin_nn(coordsT[:, :, :n_small], coordsT[:, :, :n_big])
        xn = _gather_batched(x, idx)
        skip = stack.pop()
        Wd = p['W']
        c1 = x.shape[1]
        x, dst = _mlp(xn, Wd[:, :c1], p['b'][:, None], skip=skip,
                      W2=Wd[:, c1:], scale=dscale, shift=dshift, stats=True)
        dscale, dshift = _affine_from_stats(dst, p['g'], p['be'], B * n_big)
        dr //= DEC

    x = x[:, :, jnp.argsort(perm)]

    f1 = params['fc_end'][0]
    x, st = _mlp(x, f1['W'], f1['b'][:, None], scale=dscale, shift=dshift,
                 stats=True)
    s1, h1 = _affine_from_stats(st, f1['g'], f1['be'], B * N)
    f2 = params['fc_end'][1]
    x, st = _mlp(x, f2['W'], f2['b'][:, None], scale=s1, shift=h1, stats=True)
    s2, h2 = _affine_from_stats(st, f2['g'], f2['be'], B * N)
    f3 = params['fc_end'][2]
    x = _mlp(x, f3['W'], f3['b'][:, None], scale=s2, shift=h2)
    return x


# speedup vs baseline: 8.1162x; 1.0568x over previous
import functools

import jax
import jax.numpy as jnp
from jax import lax
from jax.experimental import pallas as pl
from jax.experimental.pallas import tpu as pltpu
from jax.experimental.pallas import tpu_sc as plsc

NK = 16
DEC = 4
EPS = 1e-6
HI = lax.Precision.HIGHEST

_pcall = pl.pallas_call


def _dot(a, b):
    return lax.dot_general(a.astype(jnp.bfloat16), b.astype(jnp.bfloat16),
                           (((1,), (0,)), ((), ())),
                           preferred_element_type=jnp.float32)


def _leaky(x, s):
    return jnp.where(x >= 0, x, s * x)


def _stats_of(y, axes):
    return jnp.sum(y, axis=axes), jnp.sum(y * y, axis=axes)


def _affine_from_stats(stats, g, be, cnt):
    tot = jnp.sum(stats, axis=0)
    mean = tot[0] / cnt
    var = tot[1] / cnt - mean * mean
    scale = g / jnp.sqrt(var + EPS)
    shift = be - mean * scale
    return scale[:, None], shift[:, None]


def _d2_refstyle(cT, qT):
    r2 = jnp.sum(cT * cT, axis=0, keepdims=True)
    q2 = jnp.sum(qT * qT, axis=0, keepdims=True)
    prod = lax.dot_general(qT.astype(jnp.bfloat16), cT.astype(jnp.bfloat16),
                           (((0,), (0,)), ((), ())),
                           preferred_element_type=jnp.float32)
    return jnp.transpose(q2) + r2 - 2.0 * prod


def _knn_body(k, n, cref, qref, idx_ref, dist_ref):
    d2 = _d2_refstyle(cref[0], qref[0])
    bq = d2.shape[0]
    iota = lax.broadcasted_iota(jnp.int32, (bq, n), 1)
    for j in range(k):
        m = jnp.min(d2, axis=1, keepdims=True)
        sel = jnp.where(d2 == m, iota, n)
        ij = jnp.min(sel, axis=1, keepdims=True)
        dist_ref[0, j, :] = jnp.sqrt(jnp.maximum(m, 1e-12))[:, 0]
        idx_ref[0, j, :] = ij[:, 0]
        d2 = jnp.where(iota == ij, jnp.float32(jnp.inf), d2)


def _knn_topk(coordsT, k):
    B, _, n = coordsT.shape
    bq = min(n, 1024)
    grid = (B, n // bq)
    return _pcall(
        functools.partial(_knn_body, k, n),
        grid=grid,
        in_specs=[
            pl.BlockSpec((1, 3, n), lambda b, j: (b, 0, 0)),
            pl.BlockSpec((1, 3, bq), lambda b, j: (b, 0, j)),
        ],
        out_specs=[
            pl.BlockSpec((1, k, bq), lambda b, j: (b, 0, j)),
            pl.BlockSpec((1, k, bq), lambda b, j: (b, 0, j)),
        ],
        out_shape=[
            jax.ShapeDtypeStruct((B, k, n), jnp.int32),
            jax.ShapeDtypeStruct((B, k, n), jnp.float32),
        ],
    )(coordsT, coordsT)


def _argmin_body(nr, cref, qref, idx_ref):
    d2 = _d2_refstyle(cref[0], qref[0])
    bq = d2.shape[0]
    iota = lax.broadcasted_iota(jnp.int32, (bq, nr), 1)
    m = jnp.min(d2, axis=1, keepdims=True)
    ij = jnp.min(jnp.where(d2 == m, iota, nr), axis=1, keepdims=True)
    idx_ref[0, 0, :] = ij[:, 0]


def _argmin_nn(refT, queryT):
    B, _, nr = refT.shape
    nq = queryT.shape[2]
    bq = min(nq, 1024)
    out = _pcall(
        functools.partial(_argmin_body, nr),
        grid=(B, nq // bq),
        in_specs=[
            pl.BlockSpec((1, 3, nr), lambda b, j: (b, 0, 0)),
            pl.BlockSpec((1, 3, bq), lambda b, j: (b, 0, j)),
        ],
        out_specs=pl.BlockSpec((1, 1, bq), lambda b, j: (b, 0, j)),
        out_shape=jax.ShapeDtypeStruct((B, 1, nq), jnp.int32),
    )(refT, queryT)
    return out[:, 0, :]


def _gather_rows(table, flat_idx):
    M = flat_idx.shape[0]
    D = table.shape[1]
    Dp = ((D + 127) // 128) * 128
    if Dp != D:
        table = jnp.pad(table, ((0, 0), (0, Dp - D)))
    mesh = plsc.VectorSubcoreMesh(core_axis_name="c", subcore_axis_name="s")
    NW = 32
    Mp = M if M % (8 * NW) == 0 else ((M // (8 * NW)) + 1) * (8 * NW)
    if Mp != M:
        flat_idx = jnp.concatenate(
            [flat_idx, jnp.zeros((Mp - M,), jnp.int32)])
    b_per_w = Mp // NW
    chunk = min(128, b_per_w)
    nchunks = b_per_w // chunk
    idx2d = flat_idx.reshape(Mp // chunk, chunk)

    @functools.partial(
        pl.kernel, mesh=mesh,
        out_type=jax.ShapeDtypeStruct((Mp, Dp), jnp.float32),
        scratch_types=[
            pltpu.VMEM((nchunks, chunk), jnp.int32),
            pltpu.VMEM((chunk, Dp), jnp.float32),
            pltpu.VMEM((chunk, Dp), jnp.float32),
            pltpu.SemaphoreType.DMA,
            pltpu.SemaphoreType.DMA,
        ])
    def _k(tab_hbm, i_hbm, o_hbm, idx_v, r0, r1, s0, s1):
        wid = lax.axis_index("s") * 2 + lax.axis_index("c")
        base = wid * b_per_w
        pltpu.sync_copy(i_hbm.at[pl.ds(wid * nchunks, nchunks)], idx_v)
        if nchunks == 1:
            pltpu.async_copy(tab_hbm.at[idx_v.at[0]], r0, s0).wait()
            pltpu.sync_copy(r0, o_hbm.at[pl.ds(base, chunk)])
        else:
            pltpu.async_copy(tab_hbm.at[idx_v.at[0]], r0, s0)
            pltpu.async_copy(tab_hbm.at[idx_v.at[1]], r1, s1)

            @pl.loop(0, nchunks, step=2)
            def _(i):
                pltpu.make_async_copy(
                    o_hbm.at[pl.ds(base, chunk)], r0, s0).wait()
                pltpu.sync_copy(r0, o_hbm.at[pl.ds(base + i * chunk, chunk)])

                @pl.when(i + 2 < nchunks)
                def _():
                    pltpu.async_copy(tab_hbm.at[idx_v.at[i + 2]], r0, s0)

                pltpu.make_async_copy(
                    o_hbm.at[pl.ds(base, chunk)], r1, s1).wait()
                pltpu.sync_copy(
                    r1, o_hbm.at[pl.ds(base + (i + 1) * chunk, chunk)])

                @pl.when(i + 3 < nchunks)
                def _():
                    pltpu.async_copy(tab_hbm.at[idx_v.at[i + 3]], r1, s1)

    out = _k(table, idx2d)
    if Mp != M or Dp != D:
        out = out[:M, :D]
    return out


def _gather_batched(tblT, idx):
    B, c, R = tblT.shape
    table = tblT.transpose(0, 2, 1).reshape(B * R, c)
    off = (jnp.arange(B, dtype=jnp.int32) * R).reshape((B,) + (1,) * (idx.ndim - 1))
    flat = (idx + off).reshape(-1)
    rows = _gather_rows(table, flat)
    out = rows.reshape(idx.shape + (c,))
    perm = (0, idx.ndim) + tuple(range(1, idx.ndim))
    return out.transpose(perm)


def _enc_a_body(finalize, slope, fref, W1, b1, Wsc, bsc, fscale, fshift,
                x1_o, scp_o, st_o):
    feats = fref[0]
    if finalize:
        feats = _leaky(fscale[...] * feats + fshift[...], slope)
    x1_o[0] = _leaky(_dot(W1[...], feats) + b1[...], 0.2)
    scp = _dot(Wsc[...], feats) + bsc[...]
    scp_o[0] = scp
    s, ss = _stats_of(scp, (1,))
    st_o[0, 0, :] = s
    st_o[0, 1, :] = ss


def _enc_a(featsP, W1, b1, Wsc, bsc, fscale=None, fshift=None, slope=0.2):
    B, cin, n = featsP.shape
    d2 = W1.shape[0]
    c2 = Wsc.shape[0]
    finalize = fscale is not None
    if not finalize:
        fscale = jnp.zeros((cin, 1), jnp.float32)
        fshift = jnp.zeros((cin, 1), jnp.float32)
    full = lambda *s: pl.BlockSpec((1,) + s, lambda b: (b,) + (0,) * len(s))
    wspec = lambda a: pl.BlockSpec(a.shape, lambda b: (0,) * a.ndim)
    return _pcall(
        functools.partial(_enc_a_body, finalize, slope),
        grid=(B,),
        in_specs=[full(cin, n)] + [
            wspec(W1), wspec(b1), wspec(Wsc), wspec(bsc),
            wspec(fscale), wspec(fshift)],
        out_specs=[full(d2, n), full(c2, n), full(2, c2)],
        out_shape=[
            jax.ShapeDtypeStruct((B, d2, n), jnp.float32),
            jax.ShapeDtypeStruct((B, c2, n), jnp.float32),
            jax.ShapeDtypeStruct((B, 2, c2), jnp.float32),
        ],
    )(featsP, W1, b1, Wsc, bsc, fscale, fshift)


def _lse_body(cref, nb_r, dist_r, W_r, b_r, enc_o, st_o):
    ext = cref[0]
    W = W_r[...]
    b = b_r[...]
    K = nb_r.shape[2]
    ssum = None
    for k in range(K):
        nb = nb_r[0, :, k, :]
        dk = dist_r[0, k:k + 1, :]
        cc = jnp.concatenate([ext, nb, ext - nb, dk], axis=0)
        enc = _dot(W, cc) + b
        enc_o[0, :, k, :] = enc
        s, ss = enc, enc * enc
        if ssum is None:
            ssum, sssum = s, ss
        else:
            ssum, sssum = ssum + s, sssum + ss
    st_o[0, 0, :] = jnp.sum(ssum, axis=1)
    st_o[0, 1, :] = jnp.sum(sssum, axis=1)


def _lse(coordsT, nbg, dist, W, b):
    B, _, n = coordsT.shape
    d2 = W.shape[0]
    full = lambda *s: pl.BlockSpec((1,) + s, lambda b: (b,) + (0,) * len(s))
    wspec = lambda a: pl.BlockSpec(a.shape, lambda b: (0,) * a.ndim)
    return _pcall(
        _lse_body,
        grid=(B,),
        in_specs=[full(3, n), full(3, NK, n), full(NK, n),
                  wspec(W), wspec(b)],
        out_specs=[full(d2, NK, n), full(2, d2)],
        out_shape=[
            jax.ShapeDtypeStruct((B, d2, NK, n), jnp.float32),
            jax.ShapeDtypeStruct((B, 2, d2), jnp.float32),
        ],
    )(coordsT, nbg, dist, W, b)


def _pool_body(fin, enc_r, esc_r, esh_r, fp_r, fsc_r, fsh_r, scW_r,
               Wpa_r, Wpb_r, bp_r, p_o, st_o):
    esc = esc_r[...]
    esh = esh_r[...]
    fprev = fp_r[0]
    if fin:
        fprev = jnp.maximum(fsc_r[...] * fprev + fsh_r[...], 0.0)
    scW = scW_r[...]
    K = enc_r.shape[2]
    ench = []
    tks = []
    for k in range(K):
        e = jnp.maximum(esc * enc_r[0, :, k, :] + esh, 0.0)
        ench.append(e)
        tks.append(_dot(scW, e))
    m = tks[0]
    for k in range(1, K):
        m = jnp.maximum(m, tks[k])
    es = [jnp.exp(tks[k] - m) for k in range(K)]
    ssum = es[0]
    for k in range(1, K):
        ssum = ssum + es[k]
    attn = None
    for k in range(K):
        t = (es[k] / ssum) * ench[k]
        attn = t if attn is None else attn + t
    p = _dot(Wpa_r[...], attn) + _dot(Wpb_r[...], fprev) + bp_r[...]
    p_o[0] = p
    s, ss = _stats_of(p, (1,))
    st_o[0, 0, :] = s
    st_o[0, 1, :] = ss


def _pool(enc_pre, escale, eshift, fprev, scW, Wpa, Wpb, bp,
          fscale=None, fshift=None):
    B, d2, K, n = enc_pre.shape
    op = Wpa.shape[0]
    fin = fscale is not None
    if not fin:
        fscale = jnp.zeros((d2, 1), jnp.float32)
        fshift = jnp.zeros((d2, 1), jnp.float32)
    full = lambda *s: pl.BlockSpec((1,) + s, lambda b: (b,) + (0,) * len(s))
    wspec = lambda a: pl.BlockSpec(a.shape, lambda b: (0,) * a.ndim)
    return _pcall(
        functools.partial(_pool_body, fin),
        grid=(B,),
        in_specs=[full(d2, K, n), wspec(escale), wspec(eshift),
                  full(d2, n), wspec(fscale), wspec(fshift),
                  wspec(scW), wspec(Wpa), wspec(Wpb), wspec(bp)],
        out_specs=[full(op, n), full(2, op)],
        out_shape=[
            jax.ShapeDtypeStruct((B, op, n), jnp.float32),
            jax.ShapeDtypeStruct((B, 2, op), jnp.float32),
        ],
    )(enc_pre, escale, eshift, fprev, fscale, fshift, scW, Wpa, Wpb, bp)


def _enc_f_body(p2_r, psc_r, psh_r, Wm_r, bm_r, scp_r, ssc_r, ssh_r, out_o):
    p2 = jnp.maximum(psc_r[...] * p2_r[0] + psh_r[...], 0.0)
    y = _dot(Wm_r[...], p2) + bm_r[...] + ssc_r[...] * scp_r[0] + ssh_r[...]
    out_o[0] = _leaky(y, 0.01)


def _enc_f(p2_pre, pscale, pshift, Wm, bm, scp, sscale, sshift):
    B, d, n = p2_pre.shape
    c2 = Wm.shape[0]
    full = lambda *s: pl.BlockSpec((1,) + s, lambda b: (b,) + (0,) * len(s))
    wspec = lambda a: pl.BlockSpec(a.shape, lambda b: (0,) * a.ndim)
    return _pcall(
        _enc_f_body,
        grid=(B,),
        in_specs=[full(d, n), wspec(pscale), wspec(pshift),
                  wspec(Wm), wspec(bm), full(c2, n),
                  wspec(sscale), wspec(sshift)],
        out_specs=full(c2, n),
        out_shape=jax.ShapeDtypeStruct((B, c2, n), jnp.float32),
    )(p2_pre, pscale, pshift, Wm, bm, scp, sscale, sshift)


def _mlp_body(fin, act, stats, x_r, sc_r, sh_r, W1_r, b_r, skip_r, W2_r,
              y_o, st_o):
    x = x_r[0]
    if fin:
        x = jnp.maximum(sc_r[...] * x + sh_r[...], 0.0)
    y = _dot(W1_r[...], x) + b_r[...]
    if skip_r is not None:
        y = y + _dot(W2_r[...], skip_r[0])
    if act == "relu":
        y = jnp.maximum(y, 0.0)
    y_o[0] = y
    if stats:
        s, ss = _stats_of(y, (1,))
        st_o[0, 0, :] = s
        st_o[0, 1, :] = ss


def _mlp(x, W1, b, skip=None, W2=None, scale=None, shift=None, act=None,
         stats=False):
    B, cin, n = x.shape
    o = W1.shape[0]
    fin = scale is not None
    if not fin:
        scale = jnp.zeros((cin, 1), jnp.float32)
        shift = jnp.zeros((cin, 1), jnp.float32)
    full = lambda *s: pl.BlockSpec((1,) + s, lambda b: (b,) + (0,) * len(s))
    wspec = lambda a: pl.BlockSpec(a.shape, lambda b: (0,) * a.ndim)
    has_skip = skip is not None
    if not has_skip:
        skip = jnp.zeros((B, 8, n), jnp.float32)
        W2 = jnp.zeros((o, 8), jnp.float32)

    def body(x_r, sc_r, sh_r, W1_r, b_r, skip_r, W2_r, y_o, st_o):
        _mlp_body(fin, act, stats, x_r, sc_r, sh_r, W1_r, b_r,
                  skip_r if has_skip else None, W2_r, y_o, st_o)

    outs = _pcall(
        body,
        grid=(B,),
        in_specs=[full(cin, n), wspec(scale), wspec(shift),
                  wspec(W1), wspec(b), full(skip.shape[1], n), wspec(W2)],
        out_specs=[full(o, n), full(2, o)],
        out_shape=[
            jax.ShapeDtypeStruct((B, o, n), jnp.float32),
            jax.ShapeDtypeStruct((B, 2, o), jnp.float32),
        ],
    )(x, scale, shift, W1, b, skip, W2)
    return outs if stats else outs[0]


def _level(coordsT, featsP, p, fscale, fshift, slope):
    B, cin, n = featsP.shape
    d2 = p['mlp1']['W'].shape[0]
    d = 2 * d2

    idx, dist = _knn_topk(coordsT, NK)

    x1, scp, sc_st = _enc_a(
        featsP, p['mlp1']['W'], p['mlp1']['b'][:, None],
        p['shortcut']['W'], p['shortcut']['b'][:, None],
        fscale, fshift, slope)

    nbg = _gather_batched(coordsT, idx)

    enc1, st1 = _lse(coordsT, nbg, dist, p['lse1']['W'],
                     p['lse1']['b'][:, None])
    e1s, e1h = _affine_from_stats(st1, p['lse1']['g'], p['lse1']['be'],
                                  B * n * NK)
    scW1 = p['pool1']['score_W'][:d2, :d2]
    Wp1 = p['pool1']['mlp']['W']
    p1_pre, pst1 = _pool(enc1, e1s, e1h, x1, scW1,
                         Wp1[:, :d2], Wp1[:, d2:],
                         p['pool1']['mlp']['b'][:, None])
    p1s, p1h = _affine_from_stats(pst1, p['pool1']['mlp']['g'],
                                  p['pool1']['mlp']['be'], B * n)

    enc2, st2 = _lse(coordsT, nbg, dist, p['lse2']['W'],
                     p['lse2']['b'][:, None])
    e2s, e2h = _affine_from_stats(st2, p['lse2']['g'], p['lse2']['be'],
                                  B * n * NK)
    scW2 = p['pool2']['score_W'][:d2, :d2]
    Wp2 = p['pool2']['mlp']['W']
    p2_pre, pst2 = _pool(enc2, e2s, e2h, p1_pre, scW2,
                         Wp2[:, :d2], Wp2[:, d2:],
                         p['pool2']['mlp']['b'][:, None],
                         fscale=p1s, fshift=p1h)
    p2s, p2h = _affine_from_stats(pst2, p['pool2']['mlp']['g'],
                                  p['pool2']['mlp']['be'], B * n)

    scs, sch = _affine_from_stats(sc_st, p['shortcut']['g'],
                                  p['shortcut']['be'], B * n)
    out = _enc_f(p2_pre, p2s, p2h, p['mlp2']['W'],
                 p['mlp2']['b'][:, None], scp, scs, sch)
    return out


def kernel(input, params):
    inp = input
    B, N, _ = inp.shape
    nc = params['fc_end'][2]['W'].shape[0]

    inpT = inp.transpose(0, 2, 1)
    x0_pre, st0 = _mlp(inpT, params['fc_start']['W'],
                       params['fc_start']['b'][:, None], stats=True)
    f0s, f0h = _affine_from_stats(st0, params['bn_start']['g'],
                                  params['bn_start']['be'], B * N)

    perm = jax.random.permutation(jax.random.key(1), N)
    coordsT = inpT[:, :3, :][:, :, perm]
    x0_pre = x0_pre[:, :, perm]

    dr = 1
    x = x0_pre
    fscale, fshift, slope = f0s, f0h, 0.2
    stack = []
    for li, p in enumerate(params['enc']):
        n = N // dr
        out = _level(coordsT[:, :, :n], x[:, :, :n] if li == 0 else x,
                     p, fscale, fshift, slope)
        stack.append(out)
        dr *= DEC
        x = out[:, :, :N // dr]
        fscale = fshift = None
        slope = None

    x = _mlp(x, params['mid']['W'], params['mid']['b'][:, None], act="relu")

    dscale = dshift = None
    for p in params['dec']:
        n_small = N // dr
        n_big = DEC * N // dr
        idx = _argmin_nn(coordsT[:, :, :n_small], coordsT[:, :, :n_big])
        xn = _gather_batched(x, idx)
        skip = stack.pop()
        Wd = p['W']
        c1 = x.shape[1]
        x, dst = _mlp(xn, Wd[:, :c1], p['b'][:, None], skip=skip,
                      W2=Wd[:, c1:], scale=dscale, shift=dshift, stats=True)
        dscale, dshift = _affine_from_stats(dst, p['g'], p['be'], B * n_big)
        dr //= DEC

    x = x[:, :, jnp.argsort(perm)]

    f1 = params['fc_end'][0]
    x, st = _mlp(x, f1['W'], f1['b'][:, None], scale=dscale, shift=dshift,
                 stats=True)
    s1, h1 = _affine_from_stats(st, f1['g'], f1['be'], B * N)
    f2 = params['fc_end'][1]
    x, st = _mlp(x, f2['W'], f2['b'][:, None], scale=s1, shift=h1, stats=True)
    s2, h2 = _affine_from_stats(st, f2['g'], f2['be'], B * N)
    f3 = params['fc_end'][2]
    x = _mlp(x, f3['W'], f3['b'][:, None], scale=s2, shift=h2)
    return x
